# baseline (speedup 1.0000x reference)
import jax
import jax.numpy as jnp
from jax.experimental import pallas as pl

N = 100000
D = 128


def _relu_body(h_ref, o_ref):
    o_ref[...] = jnp.maximum(h_ref[...], 0.0)


def _relu(h):
    return pl.pallas_call(
        _relu_body,
        grid=(100,),
        in_specs=[pl.BlockSpec((N // 100, D), lambda i: (i, 0))],
        out_specs=pl.BlockSpec((N // 100, D), lambda i: (i, 0)),
        out_shape=jax.ShapeDtypeStruct((N, D), jnp.float32),
    )(h)


def kernel(x, edge_index_rel0, edge_index_rel1, edge_index_rel2, W0, b0, W1, b1, W2, b2):
    def conv(e, W, b):
        msg = jnp.take(x, e[0], axis=0)
        agg = jax.ops.segment_sum(msg, e[1], num_segments=N)
        deg = jax.ops.segment_sum(jnp.ones((e.shape[1],), x.dtype), e[1], num_segments=N)
        deg = jnp.maximum(deg, 1.0)
        return (agg / deg[:, None]) @ W + b

    h = conv(edge_index_rel0, W0, b0) + conv(edge_index_rel1, W1, b1) + conv(edge_index_rel2, W2, b2)
    return _relu(h)


# speedup vs baseline: 1.2563x; 1.2563x over previous
import functools

import jax
import jax.numpy as jnp
from jax import lax
from jax.experimental import pallas as pl
from jax.experimental.pallas import tpu as pltpu
from jax.experimental.pallas import tpu_sc as plsc

N = 100000
D = 128
E = 200000
ET = 3 * E
ETP = 600064
NT = 32
EPT = ETP // NT
PIECES = (4096, 4096, 4096, 4096, 2368)
NPR = 100352
C = 7168
NCHUNK = 42
NDP = NCHUNK * C
RPT = C // 16
DGR = C // 128
CAP = 8192
NG_ROWS = CAP // 128
SENT = 1 << 30

_mesh = plsc.VectorSubcoreMesh(
    core_axis_name="c", subcore_axis_name="s", num_cores=2, num_subcores=16
)


def _sc_body(x, srce, dste, agg_out, deg_out,
             src_p, dst_p, src2d, dst2d, rows_v, deg_v, idxg, chunk, degsp, sem):
    core = lax.axis_index("c")
    sub = lax.axis_index("s")

    z16f = jnp.zeros((16,), jnp.float32)
    z16i = jnp.zeros((16,), jnp.int32)
    iota = lax.iota(jnp.int32, 16)

    def zero_rows(r, carry):
        for j in range(8):
            rows_v[r, pl.ds(j * 16, 16)] = z16f
        return carry

    for j in range(DGR // 16):
        idxg[0, pl.ds(j * 16, 16)] = iota + (j * 16)
    idxg[0, pl.ds(DGR - 16, 16)] = iota + (DGR - 16)

    nk = NCHUNK // 2

    def chunk_body(i, carry):
        k = core + 2 * i
        lo = k * C
        base_r = sub * RPT

        lax.fori_loop(0, 128, zero_rows, 0)
        for j in range(3):
            pltpu.sync_copy(rows_v, chunk.at[pl.ds(base_r + j * 128, 128)])
        pltpu.sync_copy(rows_v.at[pl.ds(0, 64)], chunk.at[pl.ds(base_r + 384, 64)])

        @pl.when(sub < DGR // 8)
        def _zero_degsp():
            pltpu.sync_copy(rows_v.at[pl.ds(0, 8)], degsp.at[pl.ds(sub * 8, 8)])

        def zero_deg(r, c2):
            for j in range(8):
                deg_v[r, pl.ds(j * 16, 16)] = z16f
            return c2

        lax.fori_loop(0, DGR, zero_deg, 0)
        plsc.subcore_barrier()

        one16f = z16f + 1.0

        def scan_body(t, cnt):
            vs = src_p[pl.ds(t * 16, 16)]
            vd = dst_p[pl.ds(t * 16, 16)]
            m = (vd >= lo) & (vd < lo + C)
            mi = jnp.where(m, 1, 0)
            incl = plsc.cumsum(mi)
            p = (incl - mi) + cnt
            dloc = vd - lo
            plsc.store_scatter(src2d, [p >> 7, p & 127], vs, mask=m)
            plsc.store_scatter(dst2d, [p >> 7, p & 127], dloc, mask=m)
            plsc.addupdate_scatter(deg_v, [dloc >> 7, dloc & 127], one16f, mask=m)
            return cnt + jnp.max(incl)

        def flush(m_total):
            lrow = m_total >> 7
            lrow_v = z16i + lrow
            base_p = lrow << 7
            padv = z16i + C
            for j in range(8):
                colv = iota + (j * 16)
                mpad = (base_p + colv) >= m_total
                plsc.store_scatter(src2d, [lrow_v, colv], z16i, mask=mpad)
                plsc.store_scatter(dst2d, [lrow_v, colv], padv, mask=mpad)
            ng = (m_total + 127) >> 7

            def g_body(g, c2):
                pltpu.async_copy(x.at[src2d.at[g]], rows_v, sem).wait()
                pltpu.sync_copy(rows_v, chunk.at[dst2d.at[g]], add=True)
                return c2

            lax.fori_loop(0, ng, g_body, 0)

        m_total = jnp.int32(0)
        for sl in range(2):
            base_e = (sub + 16 * sl) * EPT
            off = 0
            for plen in PIECES:
                pltpu.sync_copy(srce.at[pl.ds(base_e + off, plen)],
                                src_p.at[pl.ds(0, plen)])
                pltpu.sync_copy(dste.at[pl.ds(base_e + off, plen)],
                                dst_p.at[pl.ds(0, plen)])
                m_total = lax.fori_loop(0, plen // 16, scan_body, m_total)
                off += plen
                near_full = m_total > (CAP - 4096)

                @pl.when(near_full)
                def _midflush():
                    flush(m_total)

                m_total = jnp.where(near_full, 0, m_total)
        flush(m_total)

        pltpu.sync_copy(deg_v, degsp.at[idxg.at[0]], add=True)
        plsc.subcore_barrier()

        orow = lo + base_r
        for j in range(3):
            pltpu.sync_copy(chunk.at[pl.ds(base_r + j * 128, 128)],
                            agg_out.at[pl.ds(orow + j * 128, 128)])
        pltpu.sync_copy(chunk.at[pl.ds(base_r + 384, 64)],
                        agg_out.at[pl.ds(orow + 384, 64)])

        @pl.when(sub < DGR // 8)
        def _wb_degsp():
            pltpu.sync_copy(degsp.at[pl.ds(sub * 8, 8)],
                            deg_out.at[pl.ds(k * DGR + sub * 8, 8)])
        return carry

    lax.fori_loop(0, nk, chunk_body, 0)


_sc_agg = functools.partial(
    pl.kernel,
    out_type=(
        jax.ShapeDtypeStruct((NDP, D), jnp.float32),
        jax.ShapeDtypeStruct((NDP // 128, 128), jnp.float32),
    ),
    mesh=_mesh,
    compiler_params=pltpu.CompilerParams(needs_layout_passes=False),
    scratch_types=[
        pltpu.VMEM((4096,), jnp.int32),
        pltpu.VMEM((4096,), jnp.int32),
        pltpu.VMEM((NG_ROWS, 128), jnp.int32),
        pltpu.VMEM((NG_ROWS, 128), jnp.int32),
        pltpu.VMEM((128, D), jnp.float32),
        pltpu.VMEM((DGR, 128), jnp.float32),
        pltpu.VMEM((1, DGR), jnp.int32),
        pltpu.VMEM_SHARED((C + 16, D), jnp.float32),
        pltpu.VMEM_SHARED((DGR, 128), jnp.float32),
        pltpu.SemaphoreType.DMA,
    ],
)(_sc_body)


def _tc_body(a0, a1, a2, d0, d1, d2, w0, w1, w2, bias, o):
    blk = a0.shape[0]
    i0 = lax.broadcasted_iota(jnp.int32, (blk, 8), 0)
    i1 = lax.broadcasted_iota(jnp.int32, (blk, 8), 1)
    p1 = jnp.where((i0 >> 7) == i1, 1.0, 0.0)
    j0 = lax.broadcasted_iota(jnp.int32, (blk, D), 0)
    j1 = lax.broadcasted_iota(jnp.int32, (blk, D), 1)
    q = jnp.where((j0 & 127) == j1, 1.0, 0.0)
    acc = None
    for a, dg, w in ((a0, d0, w0), (a1, d1, w1), (a2, d2, w2)):
        recip = 1.0 / jnp.maximum(dg[...], 1.0)
        s = jnp.dot(p1, recip, preferred_element_type=jnp.float32)
        rc = jnp.sum(s * q, axis=1, keepdims=True)
        t = jnp.dot(a[...] * rc, w[...], preferred_element_type=jnp.float32)
        acc = t if acc is None else acc + t
    o[...] = jnp.maximum(acc + bias[0:1, :], 0.0)


def _tc_combine(agg, deg, W0, W1, W2, bias):
    blk = 1024
    nblk = NPR // blk
    a_spec = lambda r: pl.BlockSpec((blk, D), lambda i, r=r: (r * nblk + i, 0))
    d_spec = lambda r: pl.BlockSpec((blk // 128, 128), lambda i, r=r: (r * nblk + i, 0))
    w_spec = pl.BlockSpec((D, D), lambda i: (0, 0))
    return pl.pallas_call(
        _tc_body,
        grid=(nblk,),
        in_specs=[a_spec(0), a_spec(1), a_spec(2),
                  d_spec(0), d_spec(1), d_spec(2),
                  w_spec, w_spec, w_spec,
                  pl.BlockSpec((8, D), lambda i: (0, 0))],
        out_specs=pl.BlockSpec((blk, D), lambda i: (i, 0)),
        out_shape=jax.ShapeDtypeStruct((NPR, D), jnp.float32),
    )(agg, agg, agg, deg, deg, deg, W0, W1, W2, bias)


def kernel(x, edge_index_rel0, edge_index_rel1, edge_index_rel2, W0, b0, W1, b1, W2, b2):
    src = jnp.concatenate([
        edge_index_rel0[0], edge_index_rel1[0], edge_index_rel2[0],
        jnp.zeros((ETP - ET,), jnp.int32)])
    dst = jnp.concatenate([
        edge_index_rel0[1], edge_index_rel1[1] + NPR, edge_index_rel2[1] + 2 * NPR,
        jnp.full((ETP - ET,), SENT, jnp.int32)])
    agg, deg = _sc_agg(x, src, dst)
    bias = jnp.broadcast_to((b0 + b1 + b2)[None, :], (8, D))
    out = _tc_combine(agg, deg, W0, W1, W2, bias)
    return out[:N]
